# baseline (speedup 1.0000x reference)
import jax
import jax.numpy as jnp
from jax import lax
from jax.experimental import pallas as pl
from jax.experimental.pallas import tpu as pltpu
from jax.experimental.pallas import tpu_sc as plsc

L = 2048
H = 16
TT = 4096
NC = 2
NS = 16
HSC = H // NC
KSLAB = 1
NSHIFT = 128
TW_S = 3968
QROWS = NSHIFT // NS
BLK = 8
NBUF = 2
VL = 16
NDIR = HSC - KSLAB


def _sc_body(tt_hbm, out_hbm, tts, bstage, stage, slab,
             sem_in, sem_bld, sem_sp, sem_out):
    cid = lax.axis_index("c")
    sid = lax.axis_index("s")
    hb = cid * HSC

    for j in range(HSC):
        pltpu.async_copy(
            tt_hbm.at[pl.ds((hb + j) * TT, TT)],
            tts.at[pl.ds(j * TT, TT)],
            sem_in,
        )
    for j in range(HSC):
        pltpu.make_async_copy(
            tt_hbm.at[pl.ds(0, TT)], tts.at[pl.ds(0, TT)], sem_in).wait()

    q0 = sid * QROWS
    for s in range(KSLAB):
        o0 = s * TT + (NSHIFT - 1) - q0

        @plsc.parallel_loop(0, TW_S // VL, unroll=4)
        def _(c):
            u = c * VL
            for qq in range(QROWS):
                bstage[qq, pl.ds(u, VL)] = tts[pl.ds(o0 - qq + u, VL)]

        pltpu.async_copy(
            bstage,
            slab.at[s, pl.ds(pl.multiple_of(q0, QROWS), QROWS)],
            sem_bld,
        )
    for s in range(KSLAB):
        pltpu.make_async_copy(
            bstage, slab.at[0, pl.ds(0, QROWS)], sem_bld).wait()
    plsc.subcore_barrier()

    i0s = pl.multiple_of(sid * NSHIFT, NSHIFT)
    a = pl.multiple_of((L - NSHIFT) - i0s, NSHIFT)
    for s in range(KSLAB):
        pltpu.async_copy(
            slab.at[s, :, pl.ds(a, L)],
            out_hbm.at[0, hb + s, pl.ds(i0s, NSHIFT)],
            sem_sp,
        )

    base_i = sid * NSHIFT

    def assemble(buf, hj, i0):
        o0 = hj * TT + (L - 1) - i0

        @plsc.parallel_loop(0, L // VL, unroll=4)
        def _(c):
            u = c * VL
            for rr in range(BLK):
                stage[buf, rr, pl.ds(u, VL)] = tts[pl.ds(o0 - rr + u, VL)]

    def drain_one():
        pltpu.make_async_copy(
            stage.at[0], out_hbm.at[0, 0, pl.ds(0, BLK)], sem_out).wait()

    n_pairs = NSHIFT // BLK // NBUF
    for hj in range(KSLAB, HSC):

        def loop_body(g, carry, hj=hj):
            for buf in range(NBUF):
                i0 = base_i + (g * NBUF + buf) * BLK
                if hj == KSLAB:
                    @pl.when(g > 0)
                    def _():
                        drain_one()
                else:
                    drain_one()
                assemble(buf, hj, i0)
                pltpu.async_copy(
                    stage.at[buf],
                    out_hbm.at[0, hb + hj,
                               pl.ds(pl.multiple_of(i0, BLK), BLK)],
                    sem_out,
                )
            return carry

        lax.fori_loop(0, n_pairs, loop_body, 0)
    for _ in range(NBUF):
        drain_one()

    for s in range(KSLAB):
        pltpu.make_async_copy(
            slab.at[0, :, pl.ds(0, L)],
            out_hbm.at[0, 0, pl.ds(0, NSHIFT)],
            sem_sp,
        ).wait()


@jax.jit
def _run_sc(tt):
    mesh = plsc.VectorSubcoreMesh(
        core_axis_name="c", subcore_axis_name="s", num_cores=NC, num_subcores=NS
    )
    return pl.kernel(
        _sc_body,
        out_type=jax.ShapeDtypeStruct((1, H, L, L), jnp.float32),
        mesh=mesh,
        scratch_types=[
            pltpu.VMEM((HSC * TT,), jnp.float32),
            pltpu.VMEM((QROWS, TW_S), jnp.float32),
            pltpu.VMEM((NBUF, BLK, L), jnp.float32),
            pltpu.VMEM_SHARED((KSLAB, NSHIFT, TW_S), jnp.float32),
            pltpu.SemaphoreType.DMA,
            pltpu.SemaphoreType.DMA,
            pltpu.SemaphoreType.DMA,
            pltpu.SemaphoreType.DMA,
        ],
    )(tt)


def kernel(x, bias_table):
    del x
    tt = jnp.transpose(jnp.pad(bias_table, ((0, TT - (2 * L - 1)), (0, 0))))
    return _run_sc(tt.reshape(H * TT))


# speedup vs baseline: 1.0753x; 1.0753x over previous
import jax
import jax.numpy as jnp
from jax import lax
from jax.experimental import pallas as pl
from jax.experimental.pallas import tpu as pltpu
from jax.experimental.pallas import tpu_sc as plsc

L = 2048
H = 16
TT = 4096
NC = 2
NS = 16
BLK = 16
NBUF = 2
VL = 16
ROWS_PER_W = (H * L) // (NC * NS)


def _sc_body(tt_hbm, out_hbm, tt_v, stage, sem_in, sem_out):
    cid = lax.axis_index("c")
    sid = lax.axis_index("s")
    wid = sid * NC + cid
    h = wid // 2
    half = wid % 2
    base_i = half * ROWS_PER_W

    pltpu.async_copy(tt_hbm.at[pl.ds(h * TT, TT)], tt_v, sem_in).wait()

    def assemble(buf, i0):
        o0 = (L - 1) - i0

        @plsc.parallel_loop(0, L // VL, unroll=4)
        def _(c):
            s = c * VL
            for rr in range(BLK):
                stage[buf, rr, pl.ds(s, VL)] = tt_v[pl.ds(o0 - rr + s, VL)]

    def drain_one():
        pltpu.make_async_copy(
            stage.at[0],
            out_hbm.at[0, 0, pl.ds(0, BLK)],
            sem_out,
        ).wait()

    n_pairs = ROWS_PER_W // BLK // NBUF

    def loop_body(g, carry):
        for buf in range(NBUF):
            i0 = base_i + (g * NBUF + buf) * BLK

            @pl.when(g > 0)
            def _():
                drain_one()

            assemble(buf, i0)
            pltpu.async_copy(
                stage.at[buf],
                out_hbm.at[0, h, pl.ds(pl.multiple_of(i0, BLK), BLK)],
                sem_out,
            )
        return carry

    lax.fori_loop(0, n_pairs, loop_body, 0)
    for _ in range(NBUF):
        drain_one()


@jax.jit
def _run_sc(tt):
    mesh = plsc.VectorSubcoreMesh(
        core_axis_name="c", subcore_axis_name="s", num_cores=NC, num_subcores=NS
    )
    return pl.kernel(
        _sc_body,
        out_type=jax.ShapeDtypeStruct((1, H, L, L), jnp.float32),
        mesh=mesh,
        scratch_types=[
            pltpu.VMEM((TT,), jnp.float32),
            pltpu.VMEM((NBUF, BLK, L), jnp.float32),
            pltpu.SemaphoreType.DMA,
            pltpu.SemaphoreType.DMA,
        ],
    )(tt)


def kernel(x, bias_table):
    del x
    tt = jnp.transpose(jnp.pad(bias_table, ((0, TT - (2 * L - 1)), (0, 0))))
    return _run_sc(tt.reshape(H * TT))
